# Initial kernel scaffold; baseline (speedup 1.0000x reference)
import jax
import jax.numpy as jnp
from jax.experimental import pallas as pl


def kernel(tokens, table):
    raise NotImplementedError("write your pallas kernel here")


# speedup vs baseline: 1.3005x; 1.3005x over previous
import functools
import math

import jax
import jax.numpy as jnp
from jax import lax
from jax.experimental import pallas as pl
from jax.experimental.pallas import tpu as pltpu
from jax.experimental.pallas import tpu_sc as plsc

VOCAB_D = 32
SCALE = math.sqrt(32.0)

NW = 32
CHUNK = 128
G = 10
GROUP_ROWS = G * CHUNK


def _emb_body(tok_hbm, table_hbm, out_hbm, idx_v, buf0, sem0, *, per_w):
    rows_per_w = per_w // CHUNK
    ngroups = rows_per_w // G
    wid = lax.axis_index("s") * 2 + lax.axis_index("c")
    base_row = wid * rows_per_w

    def scale_store(g, buf):
        def scale_row(r, _):
            buf[r, pl.ds(0, 16)] = buf[r, pl.ds(0, 16)] * SCALE
            buf[r, pl.ds(16, 16)] = buf[r, pl.ds(16, 16)] * SCALE
            return 0

        lax.fori_loop(0, GROUP_ROWS, scale_row, 0)

        pltpu.sync_copy(
            buf,
            out_hbm.at[pl.ds(wid * per_w + g * GROUP_ROWS, GROUP_ROWS)],
        )

    def body(g, _):
        pltpu.sync_copy(tok_hbm.at[pl.ds(base_row + g * G, G)], idx_v)
        descs = [
            pltpu.async_copy(
                table_hbm.at[idx_v.at[b]],
                buf0.at[pl.ds(b * CHUNK, CHUNK)],
                sem0,
            )
            for b in range(G)
        ]
        for d in descs:
            d.wait()
        scale_store(g, buf0)
        return 0

    lax.fori_loop(0, ngroups, body, 0)


@functools.partial(jax.jit, static_argnames=())
def kernel(tokens, table):
    nb, nt = tokens.shape
    b = nb * nt
    per_w = b // NW
    rows_per_w = per_w // CHUNK
    tok2d = tokens.reshape(NW * rows_per_w, CHUNK).astype(jnp.int32)

    mesh = plsc.VectorSubcoreMesh(core_axis_name="c", subcore_axis_name="s")
    run = pl.kernel(
        functools.partial(_emb_body, per_w=per_w),
        out_type=jax.ShapeDtypeStruct((b, VOCAB_D), jnp.float32),
        mesh=mesh,
        scratch_types=[
            pltpu.VMEM((G, CHUNK), jnp.int32),
            pltpu.VMEM((GROUP_ROWS, VOCAB_D), jnp.float32),
            pltpu.SemaphoreType.DMA,
        ],
        compiler_params=pltpu.CompilerParams(use_tc_tiling_on_sc=False),
    )
    out = run(tok2d, table)
    return out.reshape(nb, nt, VOCAB_D)


# speedup vs baseline: 1.4760x; 1.1349x over previous
import functools
import math

import jax
import jax.numpy as jnp
from jax import lax
from jax.experimental import pallas as pl
from jax.experimental.pallas import tpu as pltpu
from jax.experimental.pallas import tpu_sc as plsc

VOCAB_D = 32
SCALE = math.sqrt(32.0)

NW = 32
CHUNK = 128
G = 10
GROUP_ROWS = G * CHUNK


def _emb_body(tok_hbm, table_hbm, out_hbm, idx_v, buf0, buf1, sem0, sem1,
              *, per_w):
    rows_per_w = per_w // CHUNK
    ngroups = rows_per_w // G
    wid = lax.axis_index("s") * 2 + lax.axis_index("c")
    base_row = wid * rows_per_w

    pltpu.sync_copy(tok_hbm.at[pl.ds(base_row, rows_per_w)], idx_v)

    def fire(g, buf, sem):
        for b in range(G):
            pltpu.async_copy(
                table_hbm.at[idx_v.at[g * G + b]],
                buf.at[pl.ds(b * CHUNK, CHUNK)],
                sem,
            )

    def drain(buf, sem):
        pltpu.make_async_copy(
            table_hbm.at[pl.ds(0, GROUP_ROWS)], buf, sem
        ).wait()

    def scale_store(g, buf):
        @plsc.parallel_loop(0, GROUP_ROWS, unroll=8)
        def _(r):
            buf[r, pl.ds(0, 16)] = buf[r, pl.ds(0, 16)] * SCALE
            buf[r, pl.ds(16, 16)] = buf[r, pl.ds(16, 16)] * SCALE

        pltpu.sync_copy(
            buf,
            out_hbm.at[pl.ds(wid * per_w + g * GROUP_ROWS, GROUP_ROWS)],
        )

    npairs = ngroups // 2
    fire(0, buf0, sem0)

    def pair(p, _):
        g0 = 2 * p
        fire(g0 + 1, buf1, sem1)
        drain(buf0, sem0)
        scale_store(g0, buf0)

        @pl.when(p + 1 < npairs)
        def _():
            fire(g0 + 2, buf0, sem0)

        drain(buf1, sem1)
        scale_store(g0 + 1, buf1)
        return 0

    lax.fori_loop(0, npairs, pair, 0)


@functools.partial(jax.jit, static_argnames=())
def kernel(tokens, table):
    nb, nt = tokens.shape
    b = nb * nt
    per_w = b // NW
    rows_per_w = per_w // CHUNK
    tok2d = tokens.reshape(NW * rows_per_w, CHUNK).astype(jnp.int32)

    mesh = plsc.VectorSubcoreMesh(core_axis_name="c", subcore_axis_name="s")
    run = pl.kernel(
        functools.partial(_emb_body, per_w=per_w),
        out_type=jax.ShapeDtypeStruct((b, VOCAB_D), jnp.float32),
        mesh=mesh,
        scratch_types=[
            pltpu.VMEM((rows_per_w, CHUNK), jnp.int32),
            pltpu.VMEM((GROUP_ROWS, VOCAB_D), jnp.float32),
            pltpu.VMEM((GROUP_ROWS, VOCAB_D), jnp.float32),
            pltpu.SemaphoreType.DMA,
            pltpu.SemaphoreType.DMA,
        ],
        compiler_params=pltpu.CompilerParams(use_tc_tiling_on_sc=False),
    )
    out = run(tok2d, table)
    return out.reshape(nb, nt, VOCAB_D)


# speedup vs baseline: 1.5097x; 1.0228x over previous
import math

import jax
import jax.numpy as jnp
from jax import lax
from jax.experimental import pallas as pl
from jax.experimental.pallas import tpu as pltpu
from jax.experimental.pallas import tpu_sc as plsc

D = 32
SCALE = math.sqrt(32.0)

NW = 32
CHUNK = 128
NC4 = 4
TASK_B = 8 * CHUNK


def _emb_body(tok_hbm, table_hbm, out_hbm, idx_v, buf, buf2, sem, *,
              n_t, n_b):
    ntasks = n_t * NC4
    per_w = ntasks // NW
    wid = lax.axis_index("s") * 2 + lax.axis_index("c")
    t_words = n_b * D

    iota = lax.iota(jnp.int32, 16)

    def task(k, _):
        t = k // NC4
        c4 = k % NC4

        pltpu.sync_copy(tok_hbm.at[t, c4], idx_v)

        for ci in range(8):
            pltpu.async_copy(
                table_hbm.at[idx_v.at[ci]],
                buf.at[pl.ds(ci * CHUNK, CHUNK)],
                sem,
            )
        pltpu.make_async_copy(
            table_hbm.at[pl.ds(0, TASK_B)], buf, sem
        ).wait()

        for dh in range(2):
            dvals = dh * 16 + iota
            pb = (dvals >> 3) * 8192 + (dvals & 7) * 128

            @plsc.parallel_loop(0, TASK_B, unroll=8)
            def _(bl, pb=pb, dh=dh):
                v = buf[bl, pl.ds(dh * 16, 16)]
                pos = pb + (bl >> 7) * 1024 + (bl & 127)
                plsc.store_scatter(buf2, [pos], v * SCALE)

        for r in range(4):
            pltpu.sync_copy(
                buf2.at[pl.ds(r * 8192, 8192)],
                out_hbm.at[t, pl.ds(r * (t_words // 4) + c4 * 8192, 8192)],
            )
        return 0

    lax.fori_loop(wid * per_w, (wid + 1) * per_w, task, 0)


def kernel(tokens, table):
    n_b, n_t = tokens.shape
    tok4 = tokens.T.reshape(n_t, NC4, 8, CHUNK).astype(jnp.int32)
    t_words = n_b * D

    mesh = plsc.VectorSubcoreMesh(core_axis_name="c", subcore_axis_name="s")
    run = pl.kernel(
        lambda *a: _emb_body(*a, n_t=n_t, n_b=n_b),
        out_type=jax.ShapeDtypeStruct((n_t, t_words), jnp.float32),
        mesh=mesh,
        scratch_types=[
            pltpu.VMEM((8, CHUNK), jnp.int32),
            pltpu.VMEM((TASK_B, D), jnp.float32),
            pltpu.VMEM((TASK_B * D,), jnp.float32),
            pltpu.SemaphoreType.DMA,
        ],
        compiler_params=pltpu.CompilerParams(
            use_tc_tiling_on_sc=False, needs_layout_passes=False
        ),
    )
    out2 = run(tok4, table)
    out5 = out2.reshape(n_t, 4, n_b // CHUNK, 8, CHUNK)
    return out5.transpose(2, 4, 0, 1, 3).reshape(n_b, n_t, D)


# speedup vs baseline: 1.6817x; 1.1139x over previous
import math

import jax
import jax.numpy as jnp
from jax import lax
from jax.experimental import pallas as pl
from jax.experimental.pallas import tpu as pltpu
from jax.experimental.pallas import tpu_sc as plsc

D = 32
SCALE = math.sqrt(32.0)

NW = 32
CHUNK = 128
NCH = 8
TASK_B = 512
NSTR = TASK_B // CHUNK


def _emb_body(tok_hbm, table_hbm, out_hbm, idx_all, buf_a, buf_b, buf2_a,
              buf2_b, gsem_a, gsem_b, osem_a, osem_b, *, n_t, n_b):
    ntasks = n_t * NCH
    per_w = ntasks // NW
    wid = lax.axis_index("s") * 2 + lax.axis_index("c")
    k0 = wid * per_w
    t_words = n_b * D

    pltpu.sync_copy(tok_hbm.at[pl.ds(k0 * NSTR, per_w * NSTR)], idx_all)

    iota = lax.iota(jnp.int32, 16)
    pbs = []
    for dh in range(2):
        dvals = dh * 16 + iota
        pbs.append((dvals >> 3) * (NSTR * 1024) + (dvals & 7) * 128)

    def fire(kl, buf, gsem):
        for ci in range(NSTR):
            pltpu.async_copy(
                table_hbm.at[idx_all.at[kl * NSTR + ci]],
                buf.at[pl.ds(ci * CHUNK, CHUNK)],
                gsem,
            )

    def drain_gather(buf, gsem):
        pltpu.make_async_copy(
            table_hbm.at[pl.ds(0, TASK_B)], buf, gsem
        ).wait()

    def drain_stores(buf2, osem):
        pltpu.make_async_copy(
            out_hbm.at[0, pl.ds(0, TASK_B * D)], buf2, osem
        ).wait()

    def transpose_scale(buf, buf2):
        for dh in range(2):
            pb = pbs[dh]

            @plsc.parallel_loop(0, TASK_B, unroll=8)
            def _(bl, pb=pb, dh=dh):
                v = buf[bl, pl.ds(dh * 16, 16)]
                pos = pb + (bl >> 7) * 1024 + (bl & 127)
                plsc.store_scatter(buf2, [pos], v * SCALE)

    def stores(kl, buf2, osem):
        k = k0 + kl
        t = k // NCH
        c8 = k % NCH
        for r in range(4):
            pltpu.async_copy(
                buf2.at[pl.ds(r * (NSTR * 1024), NSTR * 1024)],
                out_hbm.at[
                    t, pl.ds(r * (t_words // 4) + c8 * (NSTR * 1024),
                             NSTR * 1024)
                ],
                osem,
            )

    npairs = per_w // 2
    fire(0, buf_a, gsem_a)

    def pair(p, _):
        kl0 = 2 * p
        fire(kl0 + 1, buf_b, gsem_b)
        drain_gather(buf_a, gsem_a)

        @pl.when(p > 0)
        def _():
            drain_stores(buf2_a, osem_a)

        transpose_scale(buf_a, buf2_a)
        stores(kl0, buf2_a, osem_a)

        @pl.when(p + 1 < npairs)
        def _():
            fire(kl0 + 2, buf_a, gsem_a)

        drain_gather(buf_b, gsem_b)

        @pl.when(p > 0)
        def _():
            drain_stores(buf2_b, osem_b)

        transpose_scale(buf_b, buf2_b)
        stores(kl0 + 1, buf2_b, osem_b)
        return 0

    lax.fori_loop(0, npairs, pair, 0)
    drain_stores(buf2_a, osem_a)
    drain_stores(buf2_b, osem_b)


def kernel(tokens, table):
    n_b, n_t = tokens.shape
    ntasks = n_t * NCH
    per_w = ntasks // NW
    tok2 = tokens.T.reshape(ntasks * NSTR, CHUNK).astype(jnp.int32)
    t_words = n_b * D

    mesh = plsc.VectorSubcoreMesh(core_axis_name="c", subcore_axis_name="s")
    run = pl.kernel(
        lambda *a: _emb_body(*a, n_t=n_t, n_b=n_b),
        out_type=jax.ShapeDtypeStruct((n_t, t_words), jnp.float32),
        mesh=mesh,
        scratch_types=[
            pltpu.VMEM((per_w * NSTR, CHUNK), jnp.int32),
            pltpu.VMEM((TASK_B, D), jnp.float32),
            pltpu.VMEM((TASK_B, D), jnp.float32),
            pltpu.VMEM((TASK_B * D,), jnp.float32),
            pltpu.VMEM((TASK_B * D,), jnp.float32),
            pltpu.SemaphoreType.DMA,
            pltpu.SemaphoreType.DMA,
            pltpu.SemaphoreType.DMA,
            pltpu.SemaphoreType.DMA,
        ],
        compiler_params=pltpu.CompilerParams(
            use_tc_tiling_on_sc=False, needs_layout_passes=False
        ),
    )
    out2 = run(tok2, table)
    out5 = out2.reshape(n_t, 4, n_b // CHUNK, 8, CHUNK)
    return out5.transpose(2, 4, 0, 1, 3).reshape(n_b, n_t, D)


# speedup vs baseline: 2.4791x; 1.4742x over previous
import math

import jax
import jax.numpy as jnp
from jax import lax
from jax.experimental import pallas as pl
from jax.experimental.pallas import tpu as pltpu
from jax.experimental.pallas import tpu_sc as plsc

D = 32
SCALE = math.sqrt(32.0)
PITCH3 = 513

NW = 32
CHUNK = 128
NCH = 8
TASK_B = 512
NSTR = TASK_B // CHUNK


def _emb_body(tok_hbm, table_hbm, out_hbm, idx_all, buf_a, buf_b, buf3,
              buf2_a, buf2_b, gsem_a, gsem_b, osem_a, osem_b, *, n_t, n_b):
    ntasks = n_t * NCH
    per_w = ntasks // NW
    wid = lax.axis_index("s") * 2 + lax.axis_index("c")
    k0 = wid * per_w
    t_words = n_b * D

    pltpu.sync_copy(tok_hbm.at[pl.ds(k0 * NSTR, per_w * NSTR)], idx_all)

    iota = lax.iota(jnp.int32, 16)

    def fire(kl, buf, gsem):
        for ci in range(NSTR):
            pltpu.async_copy(
                table_hbm.at[idx_all.at[kl * NSTR + ci]],
                buf.at[pl.ds(ci * CHUNK, CHUNK)],
                gsem,
            )

    def drain_gather(buf, gsem):
        pltpu.make_async_copy(
            table_hbm.at[pl.ds(0, TASK_B)], buf, gsem
        ).wait()

    def drain_stores(buf2, osem):
        pltpu.make_async_copy(
            out_hbm.at[0, pl.ds(0, TASK_B * D)], buf2, osem
        ).wait()

    def transpose_scale(buf, buf3, buf2):
        for dh in range(2):
            pb = (dh * 16 + iota) * PITCH3

            @plsc.parallel_loop(0, TASK_B, unroll=8)
            def _(bl, pb=pb, dh=dh):
                v = buf[bl, pl.ds(dh * 16, 16)]
                plsc.store_scatter(buf3, [pb + bl], v * SCALE)

        @plsc.parallel_loop(0, D * (TASK_B // 16), unroll=8)
        def _(i):
            d = i >> 5
            lg = i & 31
            v = buf3[pl.ds(d * PITCH3 + lg * 16, 16)]
            dst = ((d >> 3) * 4096 + (lg >> 3) * 1024
                   + (d & 7) * 128 + (lg & 7) * 16)
            buf2[pl.ds(dst, 16)] = v

    def stores(kl, buf2, osem):
        k = k0 + kl
        t = k // NCH
        c8 = k % NCH
        for r in range(4):
            pltpu.async_copy(
                buf2.at[pl.ds(r * (NSTR * 1024), NSTR * 1024)],
                out_hbm.at[
                    t, pl.ds(r * (t_words // 4) + c8 * (NSTR * 1024),
                             NSTR * 1024)
                ],
                osem,
            )

    npairs = per_w // 2
    fire(0, buf_a, gsem_a)

    def pair(p, _):
        kl0 = 2 * p
        fire(kl0 + 1, buf_b, gsem_b)
        drain_gather(buf_a, gsem_a)

        @pl.when(p > 0)
        def _():
            drain_stores(buf2_a, osem_a)

        transpose_scale(buf_a, buf3, buf2_a)
        stores(kl0, buf2_a, osem_a)

        @pl.when(p + 1 < npairs)
        def _():
            fire(kl0 + 2, buf_a, gsem_a)

        drain_gather(buf_b, gsem_b)

        @pl.when(p > 0)
        def _():
            drain_stores(buf2_b, osem_b)

        transpose_scale(buf_b, buf3, buf2_b)
        stores(kl0 + 1, buf2_b, osem_b)
        return 0

    lax.fori_loop(0, npairs, pair, 0)
    drain_stores(buf2_a, osem_a)
    drain_stores(buf2_b, osem_b)


def kernel(tokens, table):
    n_b, n_t = tokens.shape
    ntasks = n_t * NCH
    per_w = ntasks // NW
    tok2 = tokens.T.reshape(ntasks * NSTR, CHUNK).astype(jnp.int32)
    t_words = n_b * D

    mesh = plsc.VectorSubcoreMesh(core_axis_name="c", subcore_axis_name="s")
    run = pl.kernel(
        lambda *a: _emb_body(*a, n_t=n_t, n_b=n_b),
        out_type=jax.ShapeDtypeStruct((n_t, t_words), jnp.float32),
        mesh=mesh,
        scratch_types=[
            pltpu.VMEM((per_w * NSTR, CHUNK), jnp.int32),
            pltpu.VMEM((TASK_B, D), jnp.float32),
            pltpu.VMEM((TASK_B, D), jnp.float32),
            pltpu.VMEM((D * PITCH3,), jnp.float32),
            pltpu.VMEM((TASK_B * D,), jnp.float32),
            pltpu.VMEM((TASK_B * D,), jnp.float32),
            pltpu.SemaphoreType.DMA,
            pltpu.SemaphoreType.DMA,
            pltpu.SemaphoreType.DMA,
            pltpu.SemaphoreType.DMA,
        ],
        compiler_params=pltpu.CompilerParams(
            use_tc_tiling_on_sc=False, needs_layout_passes=False
        ),
    )
    out2 = run(tok2, table)
    out5 = out2.reshape(n_t, 4, n_b // CHUNK, 8, CHUNK)
    return out5.transpose(2, 4, 0, 1, 3).reshape(n_b, n_t, D)
